# baseline (device time: 117878 ns/iter reference)
import jax
import jax.numpy as jnp
from jax import lax
from jax.experimental import pallas as pl
from jax.experimental.pallas import tpu as pltpu

N_DEV = 16
B, Sq, Hq, Dh = 4, 256, 8, 128
D = Hq * Dh
SCALE = 0.08838834764831843 * 1.4426950408889634
EXT = 128
W = D + EXT
HA = 512
WB = W - HA
R = (B * Sq) // N_DEV
Skv = 1024


def kernel(x, Wq, Wo, K_ext, V_ext):
    xb = x.astype(jnp.bfloat16)
    wqb = Wq.astype(jnp.bfloat16)
    wob = Wo.astype(jnp.bfloat16)

    def body(x_ref, wq_ref, wo_ref, k_ref, v_ref, out_ref,
             q_ref, acc_ref, sta_ref, stb_ref, rsa_ref, rsb_ref,
             fin_ref, attn_ref, og_ref,
             kbuf, vbuf, ksem, vsem,
             rsa_send, rsa_recv, rsb_send, rsb_recv, ag_send, ag_recv):
        my = lax.axis_index("i")
        my_row = my * R

        def kv_copies(b, slot):
            return (
                pltpu.make_async_copy(k_ref.at[b], kbuf.at[slot],
                                      ksem.at[slot]),
                pltpu.make_async_copy(v_ref.at[b], vbuf.at[slot],
                                      vsem.at[slot]),
            )

        def rs_rdma(st, rs, ssem, rsem, c):
            return pltpu.make_async_remote_copy(
                src_ref=st.at[c],
                dst_ref=rs.at[my],
                send_sem=ssem.at[c],
                recv_sem=rsem.at[my],
                device_id=(c,),
                device_id_type=pl.DeviceIdType.MESH,
            )

        def rs_recv_rdma(st, rs, ssem, rsem, s):
            return pltpu.make_async_remote_copy(
                src_ref=st.at[s],
                dst_ref=rs.at[s],
                send_sem=ssem.at[s],
                recv_sem=rsem.at[s],
                device_id=(s,),
                device_id_type=pl.DeviceIdType.MESH,
            )

        def ag_rdma(t):
            return pltpu.make_async_remote_copy(
                src_ref=og_ref.at[pl.ds(my_row, R)],
                dst_ref=og_ref.at[pl.ds(my_row, R)],
                send_sem=ag_send.at[t],
                recv_sem=ag_recv.at[my],
                device_id=(t,),
                device_id_type=pl.DeviceIdType.MESH,
            )

        def ag_recv_rdma(s):
            return pltpu.make_async_remote_copy(
                src_ref=og_ref.at[pl.ds(s * R, R)],
                dst_ref=og_ref.at[pl.ds(s * R, R)],
                send_sem=ag_send.at[s],
                recv_sem=ag_recv.at[s],
                device_id=(s,),
                device_id_type=pl.DeviceIdType.MESH,
            )

        def reduce_and_broadcast():
            fin_ref[...] = acc_ref[pl.ds(my_row, R), :]
            for s in range(N_DEV):
                @pl.when(s != my)
                def _(s=s):
                    rs_recv_rdma(sta_ref, rsa_ref, rsa_send, rsa_recv,
                                 s).wait_recv()
                    fin_ref[:, :HA] += rsa_ref[s].astype(jnp.float32)
                    rs_recv_rdma(stb_ref, rsb_ref, rsb_send, rsb_recv,
                                 s).wait_recv()
                    fin_ref[:, HA:] += rsb_ref[s].astype(jnp.float32)
            for h in range(Hq):
                c0 = h * Dh
                attn_ref[:, c0:c0 + Dh] = (
                    fin_ref[:, c0:c0 + Dh] / fin_ref[:, D + h:D + h + 1]
                ).astype(jnp.bfloat16)
            outc = lax.dot(attn_ref[...], wo_ref[...],
                           preferred_element_type=jnp.float32)
            og_ref[pl.ds(my_row, R), :] = outc.astype(jnp.bfloat16)
            for t in range(N_DEV):
                @pl.when(t != my)
                def _(t=t):
                    ag_rdma(t).start()

        kc, vc = kv_copies(0, 0)
        kc.start()
        vc.start()
        xm = x_ref[...].reshape(B * Sq, D)
        q_ref[...] = (lax.dot(xm, wq_ref[...],
                              preferred_element_type=jnp.float32)
                      * SCALE).astype(jnp.bfloat16)

        for b in range(B):
            slot = b % 2
            r0 = b * Sq
            if b + 1 < B:
                kc2, vc2 = kv_copies(b + 1, (b + 1) % 2)
                kc2.start()
                vc2.start()
            kcw, vcw = kv_copies(b, slot)
            kcw.wait()
            vcw.wait()
            for h in range(Hq):
                c0 = h * Dh
                qbh = q_ref[r0:r0 + Sq, c0:c0 + Dh]
                kbh = kbuf[slot, :, h, :].astype(jnp.bfloat16)
                s = lax.dot_general(qbh, kbh, (((1,), (1,)), ((), ())),
                                    preferred_element_type=jnp.float32)
                p = jnp.exp2(s.astype(jnp.bfloat16))
                lvec = jnp.sum(p, axis=1, keepdims=True,
                               dtype=jnp.float32)
                o = lax.dot(p, vbuf[slot, :, h, :].astype(jnp.bfloat16),
                            preferred_element_type=jnp.float32)
                acc_ref[r0:r0 + Sq, c0:c0 + Dh] = o
                acc_ref[r0:r0 + Sq, D + h:D + h + 1] = lvec
                if h == HA // Dh - 1:
                    for j in range(4):
                        c = 4 * b + j

                        @pl.when(c != my)
                        def _(c=c):
                            sta_ref[c] = acc_ref[c * R:(c + 1) * R,
                                                 :HA].astype(jnp.bfloat16)
                            rs_rdma(sta_ref, rsa_ref, rsa_send, rsa_recv,
                                    c).start()
            acc_ref[r0:r0 + Sq, D + Hq:] = jnp.zeros(
                (Sq, EXT - Hq), jnp.float32)
            for j in range(4):
                c = 4 * b + j

                @pl.when(c != my)
                def _(c=c):
                    stb_ref[c] = acc_ref[c * R:(c + 1) * R,
                                         HA:].astype(jnp.bfloat16)
                    rs_rdma(stb_ref, rsb_ref, rsb_send, rsb_recv,
                            c).start()
            if b >= 1:
                @pl.when(my // 4 == b - 1)
                def _():
                    reduce_and_broadcast()

        @pl.when(my // 4 == B - 1)
        def _():
            reduce_and_broadcast()

        for s in range(N_DEV):
            @pl.when(s != my)
            def _(s=s):
                ag_recv_rdma(s).wait_recv()

        for c in range(N_DEV):
            @pl.when(c != my)
            def _(c=c):
                rs_rdma(sta_ref, rsa_ref, rsa_send, rsa_recv, c).wait_send()
                rs_rdma(stb_ref, rsb_ref, rsb_send, rsb_recv, c).wait_send()
                ag_rdma(c).wait_send()

        out_ref[...] = og_ref[...].astype(jnp.float32).reshape(B, Sq, D)

    return pl.pallas_call(
        body,
        out_shape=jax.ShapeDtypeStruct((B, Sq, D), jnp.float32),
        in_specs=[pl.BlockSpec(memory_space=pltpu.VMEM)] * 3
        + [pl.BlockSpec(memory_space=pl.ANY)] * 2,
        out_specs=pl.BlockSpec(memory_space=pltpu.VMEM),
        scratch_shapes=[
            pltpu.VMEM((B * Sq, D), jnp.bfloat16),
            pltpu.VMEM((B * Sq, W), jnp.float32),
            pltpu.VMEM((N_DEV, R, HA), jnp.bfloat16),
            pltpu.VMEM((N_DEV, R, WB), jnp.bfloat16),
            pltpu.VMEM((N_DEV, R, HA), jnp.bfloat16),
            pltpu.VMEM((N_DEV, R, WB), jnp.bfloat16),
            pltpu.VMEM((R, W), jnp.float32),
            pltpu.VMEM((R, D), jnp.bfloat16),
            pltpu.VMEM((B * Sq, D), jnp.bfloat16),
            pltpu.VMEM((2, Skv, Hq, Dh), jnp.float32),
            pltpu.VMEM((2, Skv, Hq, Dh), jnp.float32),
            pltpu.SemaphoreType.DMA((2,)),
            pltpu.SemaphoreType.DMA((2,)),
            pltpu.SemaphoreType.DMA((N_DEV,)),
            pltpu.SemaphoreType.DMA((N_DEV,)),
            pltpu.SemaphoreType.DMA((N_DEV,)),
            pltpu.SemaphoreType.DMA((N_DEV,)),
            pltpu.SemaphoreType.DMA((N_DEV,)),
            pltpu.SemaphoreType.DMA((N_DEV,)),
        ],
    )(xb, wqb, wob, K_ext, V_ext)


# device time: 97057 ns/iter; 1.2145x vs baseline; 1.2145x over previous
import jax
import jax.numpy as jnp
from jax import lax
from jax.experimental import pallas as pl
from jax.experimental.pallas import tpu as pltpu

N_DEV = 16
B, Sq, Hq, Dh = 4, 256, 8, 128
D = Hq * Dh
SCALE = 0.08838834764831843 * 1.4426950408889634
EXT = 128
W = D + EXT
HA = 512
WB = W - HA
R = (B * Sq) // N_DEV
Skv = 1024


def kernel(x, Wq, Wo, K_ext, V_ext):
    xb = x.astype(jnp.bfloat16)
    wqb = Wq.astype(jnp.bfloat16)
    wob = Wo.astype(jnp.bfloat16)

    def body(x_ref, wq_ref, wo_ref, k_ref, v_ref, out_ref,
             q_ref, acc_ref, sta_ref, stb_ref, rsa_ref, rsb_ref,
             fin_ref, attn_ref, og_ref,
             kbuf, vbuf, ksem, vsem,
             rsa_send, rsa_recv, rsb_send, rsb_recv, ag_send, ag_recv):
        my = lax.axis_index("i")
        my_row = my * R

        def kv_copies(b, slot):
            return (
                pltpu.make_async_copy(k_ref.at[b], kbuf.at[slot],
                                      ksem.at[slot]),
                pltpu.make_async_copy(v_ref.at[b], vbuf.at[slot],
                                      vsem.at[slot]),
            )

        def rs_rdma(st, rs, ssem, rsem, c):
            return pltpu.make_async_remote_copy(
                src_ref=st.at[c],
                dst_ref=rs.at[my],
                send_sem=ssem.at[c],
                recv_sem=rsem.at[my],
                device_id=(c,),
                device_id_type=pl.DeviceIdType.MESH,
            )

        def rs_recv_rdma(st, rs, ssem, rsem, s):
            return pltpu.make_async_remote_copy(
                src_ref=st.at[s],
                dst_ref=rs.at[s],
                send_sem=ssem.at[s],
                recv_sem=rsem.at[s],
                device_id=(s,),
                device_id_type=pl.DeviceIdType.MESH,
            )

        def ag_rdma(t):
            return pltpu.make_async_remote_copy(
                src_ref=og_ref.at[pl.ds(my_row, R)],
                dst_ref=og_ref.at[pl.ds(my_row, R)],
                send_sem=ag_send.at[t],
                recv_sem=ag_recv.at[my],
                device_id=(t,),
                device_id_type=pl.DeviceIdType.MESH,
            )

        def ag_recv_rdma(s):
            return pltpu.make_async_remote_copy(
                src_ref=og_ref.at[pl.ds(s * R, R)],
                dst_ref=og_ref.at[pl.ds(s * R, R)],
                send_sem=ag_send.at[s],
                recv_sem=ag_recv.at[s],
                device_id=(s,),
                device_id_type=pl.DeviceIdType.MESH,
            )

        kc, vc = kv_copies(0, 0)
        kc.start()
        vc.start()
        xm = x_ref[...].reshape(B * Sq, D)
        q_ref[...] = (lax.dot(xm, wq_ref[...],
                              preferred_element_type=jnp.float32)
                      * SCALE).astype(jnp.bfloat16)

        for b in range(B):
            slot = b % 2
            r0 = b * Sq
            if b + 1 < B:
                kc2, vc2 = kv_copies(b + 1, (b + 1) % 2)
                kc2.start()
                vc2.start()
            kcw, vcw = kv_copies(b, slot)
            kcw.wait()
            vcw.wait()
            for h in range(Hq):
                c0 = h * Dh
                qbh = q_ref[r0:r0 + Sq, c0:c0 + Dh]
                kbh = kbuf[slot, :, h, :].astype(jnp.bfloat16)
                s = lax.dot_general(qbh, kbh, (((1,), (1,)), ((), ())),
                                    preferred_element_type=jnp.float32)
                p = jnp.exp2(s.astype(jnp.bfloat16))
                lvec = jnp.sum(p, axis=1, keepdims=True,
                               dtype=jnp.float32)
                o = lax.dot(p, vbuf[slot, :, h, :].astype(jnp.bfloat16),
                            preferred_element_type=jnp.float32)
                acc_ref[r0:r0 + Sq, c0:c0 + Dh] = o
                acc_ref[r0:r0 + Sq, D + h:D + h + 1] = lvec
                if h == HA // Dh - 1:
                    for j in range(4):
                        c = 4 * b + j

                        @pl.when(c != my)
                        def _(c=c):
                            sta_ref[c] = acc_ref[c * R:(c + 1) * R,
                                                 :HA].astype(jnp.bfloat16)
                            rs_rdma(sta_ref, rsa_ref, rsa_send, rsa_recv,
                                    c).start()
            acc_ref[r0:r0 + Sq, D + Hq:] = jnp.zeros(
                (Sq, EXT - Hq), jnp.float32)
            for j in range(4):
                c = 4 * b + j

                @pl.when(c != my)
                def _(c=c):
                    stb_ref[c] = acc_ref[c * R:(c + 1) * R,
                                         HA:].astype(jnp.bfloat16)
                    rs_rdma(stb_ref, rsb_ref, rsb_send, rsb_recv,
                            c).start()

        fin_ref[...] = acc_ref[pl.ds(my_row, R), :]
        for s in range(N_DEV):
            @pl.when(s != my)
            def _(s=s):
                rs_recv_rdma(sta_ref, rsa_ref, rsa_send, rsa_recv,
                             s).wait_recv()
                fin_ref[:, :HA] += rsa_ref[s].astype(jnp.float32)
                rs_recv_rdma(stb_ref, rsb_ref, rsb_send, rsb_recv,
                             s).wait_recv()
                fin_ref[:, HA:] += rsb_ref[s].astype(jnp.float32)

        for h in range(Hq):
            c0 = h * Dh
            attn_ref[:, c0:c0 + Dh] = (
                fin_ref[:, c0:c0 + Dh] / fin_ref[:, D + h:D + h + 1]
            ).astype(jnp.bfloat16)
        outc = lax.dot(attn_ref[...], wo_ref[...],
                       preferred_element_type=jnp.float32)
        og_ref[pl.ds(my_row, R), :] = outc.astype(jnp.bfloat16)

        for t in range(N_DEV):
            @pl.when(t != my)
            def _(t=t):
                ag_rdma(t).start()
        for s in range(N_DEV):
            @pl.when(s != my)
            def _(s=s):
                ag_recv_rdma(s).wait_recv()

        for c in range(N_DEV):
            @pl.when(c != my)
            def _(c=c):
                rs_rdma(sta_ref, rsa_ref, rsa_send, rsa_recv, c).wait_send()
                rs_rdma(stb_ref, rsb_ref, rsb_send, rsb_recv, c).wait_send()
                ag_rdma(c).wait_send()

        out_ref[...] = og_ref[...].astype(jnp.float32).reshape(B, Sq, D)

    return pl.pallas_call(
        body,
        out_shape=jax.ShapeDtypeStruct((B, Sq, D), jnp.float32),
        in_specs=[pl.BlockSpec(memory_space=pltpu.VMEM)] * 3
        + [pl.BlockSpec(memory_space=pl.ANY)] * 2,
        out_specs=pl.BlockSpec(memory_space=pltpu.VMEM),
        scratch_shapes=[
            pltpu.VMEM((B * Sq, D), jnp.bfloat16),
            pltpu.VMEM((B * Sq, W), jnp.float32),
            pltpu.VMEM((N_DEV, R, HA), jnp.bfloat16),
            pltpu.VMEM((N_DEV, R, WB), jnp.bfloat16),
            pltpu.VMEM((N_DEV, R, HA), jnp.bfloat16),
            pltpu.VMEM((N_DEV, R, WB), jnp.bfloat16),
            pltpu.VMEM((R, W), jnp.float32),
            pltpu.VMEM((R, D), jnp.bfloat16),
            pltpu.VMEM((B * Sq, D), jnp.bfloat16),
            pltpu.VMEM((2, Skv, Hq, Dh), jnp.float32),
            pltpu.VMEM((2, Skv, Hq, Dh), jnp.float32),
            pltpu.SemaphoreType.DMA((2,)),
            pltpu.SemaphoreType.DMA((2,)),
            pltpu.SemaphoreType.DMA((N_DEV,)),
            pltpu.SemaphoreType.DMA((N_DEV,)),
            pltpu.SemaphoreType.DMA((N_DEV,)),
            pltpu.SemaphoreType.DMA((N_DEV,)),
            pltpu.SemaphoreType.DMA((N_DEV,)),
            pltpu.SemaphoreType.DMA((N_DEV,)),
        ],
    )(xb, wqb, wob, K_ext, V_ext)
